# baseline (speedup 1.0000x reference)
import jax
import jax.numpy as jnp
from jax.experimental import pallas as pl
from jax.experimental.pallas import tpu as pltpu

_P = 64
_K = 1024
_D = 64


def _vq_body(z_ref, wt_ref, w_ref, out_ref):
    for p0 in range(0, _P, 8):
        z = z_ref[p0:p0 + 8, :]
        acc = None
        for c in range(8):
            pairs = []
            for s in range(4):
                d = 8 * c + s
                da = z[:, d:d + 1] - wt_ref[d:d + 1, :]
                ta = da * da
                db = z[:, d + 4:d + 5] - wt_ref[d + 4:d + 5, :]
                tb = db * db
                pairs.append(ta + tb)
            quads = [pairs[0] + pairs[2], pairs[1] + pairs[3]]
            oct_c = quads[0] + quads[1]
            acc = oct_c if acc is None else acc + oct_c
        dist = jnp.sqrt(acc)
        m = jnp.min(dist, axis=1, keepdims=True)
        kidx = jax.lax.broadcasted_iota(jnp.int32, dist.shape, 1)
        cand = jnp.where(dist == m, kidx, jnp.int32(_K))
        amin = jnp.min(cand, axis=1, keepdims=True)
        onehot = (kidx == amin).astype(jnp.float32)
        q = jax.lax.dot_general(
            onehot, w_ref[...], (((1,), (0,)), ((), ())),
            preferred_element_type=jnp.float32,
            precision=jax.lax.Precision.HIGHEST)
        out_ref[p0:p0 + 8, :] = z + (q - z)


def kernel(latents, weight):
    B, D, H, W = latents.shape
    z = jnp.moveaxis(latents, 1, -1).reshape(-1, D)
    P = z.shape[0]
    wt = weight.T
    out_rows = pl.pallas_call(
        _vq_body,
        grid=(P // _P,),
        in_specs=[
            pl.BlockSpec((_P, D), lambda i: (i, 0)),
            pl.BlockSpec((D, _K), lambda i: (0, 0)),
            pl.BlockSpec((_K, D), lambda i: (0, 0)),
        ],
        out_specs=pl.BlockSpec((_P, D), lambda i: (i, 0)),
        out_shape=jax.ShapeDtypeStruct((P, D), jnp.float32),
        compiler_params=pltpu.CompilerParams(
            dimension_semantics=("parallel",)),
    )(z, wt, weight)
    return jnp.moveaxis(out_rows.reshape(B, H, W, D), -1, 1)


# speedup vs baseline: 1.8130x; 1.8130x over previous
import jax
import jax.numpy as jnp
from jax.experimental import pallas as pl
from jax.experimental.pallas import tpu as pltpu

_P = 64
_K = 1024
_D = 64


def _vq_body(z_ref, wt_ref, w_ref, out_ref):
    z = z_ref[...]
    acc = None
    for c in range(8):
        pairs = []
        for s in range(4):
            d = 8 * c + s
            da = z[:, d:d + 1] - wt_ref[d:d + 1, :]
            ta = da * da
            db = z[:, d + 4:d + 5] - wt_ref[d + 4:d + 5, :]
            tb = db * db
            pairs.append(ta + tb)
        quads = [pairs[0] + pairs[2], pairs[1] + pairs[3]]
        oct_c = quads[0] + quads[1]
        acc = oct_c if acc is None else acc + oct_c
    dist = jnp.sqrt(acc)
    m = jnp.min(dist, axis=1, keepdims=True)
    kidx = jax.lax.broadcasted_iota(jnp.int32, dist.shape, 1)
    cand = jnp.where(dist == m, kidx, jnp.int32(_K))
    amin = jnp.min(cand, axis=1, keepdims=True)
    onehot = (kidx == amin).astype(jnp.float32)
    q = jax.lax.dot_general(
        onehot, w_ref[...], (((1,), (0,)), ((), ())),
        preferred_element_type=jnp.float32,
        precision=jax.lax.Precision.HIGHEST)
    out_ref[...] = z + (q - z)


def kernel(latents, weight):
    B, D, H, W = latents.shape
    z = jnp.moveaxis(latents, 1, -1).reshape(-1, D)
    P = z.shape[0]
    wt = weight.T
    out_rows = pl.pallas_call(
        _vq_body,
        grid=(P // _P,),
        in_specs=[
            pl.BlockSpec((_P, D), lambda i: (i, 0)),
            pl.BlockSpec((D, _K), lambda i: (0, 0)),
            pl.BlockSpec((_K, D), lambda i: (0, 0)),
        ],
        out_specs=pl.BlockSpec((_P, D), lambda i: (i, 0)),
        out_shape=jax.ShapeDtypeStruct((P, D), jnp.float32),
        compiler_params=pltpu.CompilerParams(
            dimension_semantics=("parallel",)),
    )(z, wt, weight)
    return jnp.moveaxis(out_rows.reshape(B, H, W, D), -1, 1)


# speedup vs baseline: 2.0525x; 1.1321x over previous
import jax
import jax.numpy as jnp
from jax.experimental import pallas as pl
from jax.experimental.pallas import tpu as pltpu

_P = 128
_K = 1024
_D = 64


def _vq_body(z_ref, wt_ref, w_ref, out_ref):
    z = z_ref[...]
    acc = None
    for c in range(8):
        pairs = []
        for s in range(4):
            d = 8 * c + s
            da = z[:, d:d + 1] - wt_ref[d:d + 1, :]
            ta = da * da
            db = z[:, d + 4:d + 5] - wt_ref[d + 4:d + 5, :]
            tb = db * db
            pairs.append(ta + tb)
        quads = [pairs[0] + pairs[2], pairs[1] + pairs[3]]
        oct_c = quads[0] + quads[1]
        acc = oct_c if acc is None else acc + oct_c
    dist = jnp.sqrt(acc)
    m = jnp.min(dist, axis=1, keepdims=True)
    kidx = jax.lax.broadcasted_iota(jnp.int32, dist.shape, 1)
    cand = jnp.where(dist == m, kidx, jnp.int32(_K))
    amin = jnp.min(cand, axis=1, keepdims=True)
    onehot = (kidx == amin).astype(jnp.float32)
    q = jax.lax.dot_general(
        onehot, w_ref[...], (((1,), (0,)), ((), ())),
        preferred_element_type=jnp.float32,
        precision=jax.lax.Precision.HIGHEST)
    out_ref[...] = z + (q - z)


def kernel(latents, weight):
    B, D, H, W = latents.shape
    z = jnp.moveaxis(latents, 1, -1).reshape(-1, D)
    P = z.shape[0]
    wt = weight.T
    out_rows = pl.pallas_call(
        _vq_body,
        grid=(P // _P,),
        in_specs=[
            pl.BlockSpec((_P, D), lambda i: (i, 0)),
            pl.BlockSpec((D, _K), lambda i: (0, 0)),
            pl.BlockSpec((_K, D), lambda i: (0, 0)),
        ],
        out_specs=pl.BlockSpec((_P, D), lambda i: (i, 0)),
        out_shape=jax.ShapeDtypeStruct((P, D), jnp.float32),
        compiler_params=pltpu.CompilerParams(
            dimension_semantics=("parallel",)),
    )(z, wt, weight)
    return jnp.moveaxis(out_rows.reshape(B, H, W, D), -1, 1)


# speedup vs baseline: 2.1518x; 1.0484x over previous
import jax
import jax.numpy as jnp
from jax.experimental import pallas as pl
from jax.experimental.pallas import tpu as pltpu

_P = 256
_K = 1024
_D = 64


def _vq_body(z_ref, wt_ref, w_ref, out_ref):
    z = z_ref[...]
    acc = None
    for c in range(8):
        pairs = []
        for s in range(4):
            d = 8 * c + s
            da = z[:, d:d + 1] - wt_ref[d:d + 1, :]
            ta = da * da
            db = z[:, d + 4:d + 5] - wt_ref[d + 4:d + 5, :]
            tb = db * db
            pairs.append(ta + tb)
        quads = [pairs[0] + pairs[2], pairs[1] + pairs[3]]
        oct_c = quads[0] + quads[1]
        acc = oct_c if acc is None else acc + oct_c
    dist = jnp.sqrt(acc)
    m = jnp.min(dist, axis=1, keepdims=True)
    kidx = jax.lax.broadcasted_iota(jnp.int32, dist.shape, 1)
    cand = jnp.where(dist == m, kidx, jnp.int32(_K))
    amin = jnp.min(cand, axis=1, keepdims=True)
    onehot = (kidx == amin).astype(jnp.float32)
    q = jax.lax.dot_general(
        onehot, w_ref[...], (((1,), (0,)), ((), ())),
        preferred_element_type=jnp.float32,
        precision=jax.lax.Precision.HIGHEST)
    out_ref[...] = z + (q - z)


def kernel(latents, weight):
    B, D, H, W = latents.shape
    z = jnp.moveaxis(latents, 1, -1).reshape(-1, D)
    P = z.shape[0]
    wt = weight.T
    out_rows = pl.pallas_call(
        _vq_body,
        grid=(P // _P,),
        in_specs=[
            pl.BlockSpec((_P, D), lambda i: (i, 0)),
            pl.BlockSpec((D, _K), lambda i: (0, 0)),
            pl.BlockSpec((_K, D), lambda i: (0, 0)),
        ],
        out_specs=pl.BlockSpec((_P, D), lambda i: (i, 0)),
        out_shape=jax.ShapeDtypeStruct((P, D), jnp.float32),
        compiler_params=pltpu.CompilerParams(
            dimension_semantics=("parallel",)),
    )(z, wt, weight)
    return jnp.moveaxis(out_rows.reshape(B, H, W, D), -1, 1)
